# baseline (speedup 1.0000x reference)
import functools
import jax
import jax.numpy as jnp
from jax import lax
from jax.experimental import pallas as pl
from jax.experimental.pallas import tpu as pltpu
from jax.experimental.pallas import tpu_sc as plsc

IMG = 64
NU = 64
SHEET = IMG * NU
NB = 4
NSL = SHEET // 128
TC_BANDS = 48
SC_BANDS = NU - TC_BANDS
SC_ROWS = 32
SC_CH = 4


def _tc_body(x2_ref, h_ref, som_ref, var_ref, out_ref, s_ref):
    i = pl.program_id(0)
    x2 = x2_ref[...]
    for c in range(NSL):
        som4 = som_ref[:, c * 128:(c + 1) * 128].reshape(NB, IMG, 128)
        var4 = var_ref[:, c * 128:(c + 1) * 128].reshape(NB, IMG, 128)
        d = som4 - x2[None]
        e = (d * d) / var4
        s_ref[c * NB:(c + 1) * NB, :] = jnp.sum(e, axis=1)
    dists = jnp.dot(s_ref[...], h_ref[...], preferred_element_type=jnp.float32)
    m = jnp.min(dists)

    @pl.when(i == 0)
    def _():
        out_ref[0, 0] = m

    @pl.when(i > 0)
    def _():
        out_ref[0, 0] = jnp.minimum(out_ref[0, 0], m)


def _tc_part(som, running_variance, x):
    x2 = jnp.tile(x, (1, 2))
    hr = lax.broadcasted_iota(jnp.int32, (128, 2), 0) // IMG
    hc = lax.broadcasted_iota(jnp.int32, (128, 2), 1)
    h = (hr == hc).astype(jnp.float32)
    res = pl.pallas_call(
        _tc_body,
        grid=(TC_BANDS // NB,),
        in_specs=[
            pl.BlockSpec((IMG, 128), lambda i: (0, 0)),
            pl.BlockSpec((128, 2), lambda i: (0, 0)),
            pl.BlockSpec((NB * IMG, SHEET), lambda i: (i, 0)),
            pl.BlockSpec((NB * IMG, SHEET), lambda i: (i, 0)),
        ],
        out_specs=pl.BlockSpec(memory_space=pltpu.SMEM),
        out_shape=jax.ShapeDtypeStruct((1, 1), jnp.float32),
        scratch_shapes=[pltpu.VMEM((NSL * NB, 128), jnp.float32)],
    )(x2, h, som, running_variance)
    return res


_SC_MESH = plsc.VectorSubcoreMesh(core_axis_name="c", subcore_axis_name="s")


@functools.partial(
    pl.kernel,
    out_type=jax.ShapeDtypeStruct((16 * SC_BANDS, SHEET), jnp.float32),
    mesh=_SC_MESH,
    scratch_types=[
        pltpu.VMEM((IMG, IMG), jnp.float32),
        pltpu.VMEM((2, SC_CH, SHEET), jnp.float32),
        pltpu.VMEM((2, SC_CH, SHEET), jnp.float32),
        pltpu.VMEM((SHEET,), jnp.float32),
        pltpu.SemaphoreType.DMA((2,)),
        pltpu.SemaphoreType.DMA((2,)),
    ],
)
def _sc_kernel(som_hbm, var_hbm, x_hbm, out_hbm,
               x_v, som_v, var_v, acc_v, sem_s, sem_v):
    c = lax.axis_index("c")
    s = lax.axis_index("s")
    w = c * 16 + s
    r0 = TC_BANDS * IMG + w * SC_ROWS
    xbase = (w % 2) * SC_ROWS
    pltpu.sync_copy(x_hbm, x_v)

    nchunk = SC_ROWS // SC_CH

    def dma(chunk):
        b = chunk % 2
        return (
            pltpu.make_async_copy(
                som_hbm.at[pl.ds(r0 + chunk * SC_CH, SC_CH), :], som_v.at[b], sem_s.at[b]
            ),
            pltpu.make_async_copy(
                var_hbm.at[pl.ds(r0 + chunk * SC_CH, SC_CH), :], var_v.at[b], sem_v.at[b]
            ),
        )

    for h in dma(0):
        h.start()
    for chunk in range(nchunk):
        for h in dma(chunk):
            h.wait()
        if chunk + 1 < nchunk:
            for h in dma(chunk + 1):
                h.start()
        b = chunk % 2

        @functools.partial(plsc.parallel_loop, 0, NU, unroll=2)
        def _(jq, chunk=chunk, b=b):
            base = jq * 64
            for k in range(4):
                col = base + k * 16
                a = jnp.zeros((16,), jnp.float32)
                for r in range(SC_CH):
                    sv = som_v[b, r, pl.ds(col, 16)]
                    vv = var_v[b, r, pl.ds(col, 16)]
                    xr = xbase + chunk * SC_CH + r
                    d = sv - x_v[xr, pl.ds(k * 16, 16)]
                    a = a + d * d / vv
                acc_v[pl.ds(col, 16)] = a

        pltpu.sync_copy(acc_v, out_hbm.at[w * nchunk + chunk])


def _epi_body(tc_ref, g_ref, a_ref, out_ref):
    p = a_ref[...].reshape(SC_BANDS, 16, SHEET).sum(axis=1)
    dists = jnp.dot(p, g_ref[...], preferred_element_type=jnp.float32)
    out_ref[0, 0] = jnp.minimum(tc_ref[0, 0], jnp.min(dists))


@jax.jit
def kernel(som, running_variance, x):
    tc_min = _tc_part(som, running_variance, x)
    sc_acc = _sc_kernel(som, running_variance, x)
    gr = lax.broadcasted_iota(jnp.int32, (SHEET, NU), 0) // IMG
    gc = lax.broadcasted_iota(jnp.int32, (SHEET, NU), 1)
    g = (gr == gc).astype(jnp.float32)
    res = pl.pallas_call(
        _epi_body,
        in_specs=[
            pl.BlockSpec(memory_space=pltpu.SMEM),
            pl.BlockSpec((SHEET, NU), lambda: (0, 0)),
            pl.BlockSpec((16 * SC_BANDS, SHEET), lambda: (0, 0)),
        ],
        out_specs=pl.BlockSpec(memory_space=pltpu.SMEM),
        out_shape=jax.ShapeDtypeStruct((1, 1), jnp.float32),
    )(tc_min, g, sc_acc)
    return res[0, 0]


# speedup vs baseline: 1.0160x; 1.0160x over previous
import functools
import jax
import jax.numpy as jnp
from jax import lax
from jax.experimental import pallas as pl
from jax.experimental.pallas import tpu as pltpu
from jax.experimental.pallas import tpu_sc as plsc

IMG = 64
NU = 64
SHEET = IMG * NU
NB = 4
NSL = SHEET // 128
TC_BANDS = 56
SC_BANDS = NU - TC_BANDS
SC_ROWS = SC_BANDS * IMG // 32
SC_CH = 4
SC_NCH = SC_ROWS // SC_CH
SC_OUT_ROWS = 32 * SC_NCH


def _tc_body(x2_ref, h_ref, som_ref, var_ref, out_ref, s_ref):
    i = pl.program_id(0)
    x2 = x2_ref[...]
    for c in range(NSL):
        som4 = som_ref[:, c * 128:(c + 1) * 128].reshape(NB, IMG, 128)
        var4 = var_ref[:, c * 128:(c + 1) * 128].reshape(NB, IMG, 128)
        d = som4 - x2[None]
        e = (d * d) / var4
        s_ref[c * NB:(c + 1) * NB, :] = jnp.sum(e, axis=1)
    dists = jnp.dot(s_ref[...], h_ref[...], preferred_element_type=jnp.float32)
    m = jnp.min(dists)

    @pl.when(i == 0)
    def _():
        out_ref[0, 0] = m

    @pl.when(i > 0)
    def _():
        out_ref[0, 0] = jnp.minimum(out_ref[0, 0], m)


def _tc_part(som, running_variance, x):
    x2 = jnp.tile(x, (1, 2))
    hr = lax.broadcasted_iota(jnp.int32, (128, 2), 0) // IMG
    hc = lax.broadcasted_iota(jnp.int32, (128, 2), 1)
    h = (hr == hc).astype(jnp.float32)
    res = pl.pallas_call(
        _tc_body,
        grid=(TC_BANDS // NB,),
        in_specs=[
            pl.BlockSpec((IMG, 128), lambda i: (0, 0)),
            pl.BlockSpec((128, 2), lambda i: (0, 0)),
            pl.BlockSpec((NB * IMG, SHEET), lambda i: (i, 0)),
            pl.BlockSpec((NB * IMG, SHEET), lambda i: (i, 0)),
        ],
        out_specs=pl.BlockSpec(memory_space=pltpu.SMEM),
        out_shape=jax.ShapeDtypeStruct((1, 1), jnp.float32),
        scratch_shapes=[pltpu.VMEM((NSL * NB, 128), jnp.float32)],
    )(x2, h, som, running_variance)
    return res


_SC_MESH = plsc.VectorSubcoreMesh(core_axis_name="c", subcore_axis_name="s")


@functools.partial(
    pl.kernel,
    out_type=jax.ShapeDtypeStruct((SC_OUT_ROWS, SHEET), jnp.float32),
    mesh=_SC_MESH,
    scratch_types=[
        pltpu.VMEM((IMG, IMG), jnp.float32),
        pltpu.VMEM((SC_CH, SHEET), jnp.float32),
        pltpu.VMEM((SC_CH, SHEET), jnp.float32),
        pltpu.VMEM((SHEET,), jnp.float32),
    ],
)
def _sc_kernel(som_hbm, var_hbm, x_hbm, out_hbm, x_v, som_v, var_v, acc_v):
    c = lax.axis_index("c")
    s = lax.axis_index("s")
    w = c * 16 + s
    r0 = TC_BANDS * IMG + w * SC_ROWS
    xbase = (w * SC_ROWS) % IMG
    pltpu.sync_copy(x_hbm, x_v)

    for chunk in range(SC_NCH):
        pltpu.sync_copy(som_hbm.at[pl.ds(r0 + chunk * SC_CH, SC_CH), :], som_v)
        pltpu.sync_copy(var_hbm.at[pl.ds(r0 + chunk * SC_CH, SC_CH), :], var_v)

        @functools.partial(plsc.parallel_loop, 0, NU, unroll=2)
        def _(jq, chunk=chunk):
            base = jq * 64
            for k in range(4):
                col = base + k * 16
                a = jnp.zeros((16,), jnp.float32)
                for r in range(SC_CH):
                    sv = som_v[r, pl.ds(col, 16)]
                    vv = var_v[r, pl.ds(col, 16)]
                    xr = xbase + chunk * SC_CH + r
                    d = sv - x_v[xr, pl.ds(k * 16, 16)]
                    a = a + d * d / vv
                acc_v[pl.ds(col, 16)] = a

        pltpu.sync_copy(acc_v, out_hbm.at[w * SC_NCH + chunk])


def _epi_body(tc_ref, g_ref, a_ref, out_ref):
    p = a_ref[...].reshape(SC_BANDS, 16, SHEET).sum(axis=1)
    dists = jnp.dot(p, g_ref[...], preferred_element_type=jnp.float32)
    out_ref[0, 0] = jnp.minimum(tc_ref[0, 0], jnp.min(dists))


@jax.jit
def kernel(som, running_variance, x):
    tc_min = _tc_part(som, running_variance, x)
    sc_acc = _sc_kernel(som, running_variance, x)
    gr = lax.broadcasted_iota(jnp.int32, (SHEET, NU), 0) // IMG
    gc = lax.broadcasted_iota(jnp.int32, (SHEET, NU), 1)
    g = (gr == gc).astype(jnp.float32)
    res = pl.pallas_call(
        _epi_body,
        in_specs=[
            pl.BlockSpec(memory_space=pltpu.SMEM),
            pl.BlockSpec((SHEET, NU), lambda: (0, 0)),
            pl.BlockSpec((SC_OUT_ROWS, SHEET), lambda: (0, 0)),
        ],
        out_specs=pl.BlockSpec(memory_space=pltpu.SMEM),
        out_shape=jax.ShapeDtypeStruct((1, 1), jnp.float32),
    )(tc_min, g, sc_acc)
    return res[0, 0]


# speedup vs baseline: 1.5614x; 1.5369x over previous
import jax
import jax.numpy as jnp
from jax import lax
from jax.experimental import pallas as pl
from jax.experimental.pallas import tpu as pltpu

IMG = 64
NU = 64
SHEET = IMG * NU
NB = 4
NSL = SHEET // 128


def _tc_body(x2_ref, h_ref, som_ref, var_ref, out_ref, s_ref):
    i = pl.program_id(0)
    x2 = x2_ref[...]
    for c in range(NSL):
        som4 = som_ref[:, c * 128:(c + 1) * 128].reshape(NB, IMG, 128)
        var4 = var_ref[:, c * 128:(c + 1) * 128].reshape(NB, IMG, 128)
        d = som4 - x2[None]
        e = (d * d) / var4
        s_ref[c * NB:(c + 1) * NB, :] = jnp.sum(e, axis=1)
    dists = jnp.dot(s_ref[...], h_ref[...], preferred_element_type=jnp.float32)
    m = jnp.min(dists)

    @pl.when(i == 0)
    def _():
        out_ref[0, 0] = m

    @pl.when(i > 0)
    def _():
        out_ref[0, 0] = jnp.minimum(out_ref[0, 0], m)


@jax.jit
def kernel(som, running_variance, x):
    x2 = jnp.tile(x, (1, 2))
    hr = lax.broadcasted_iota(jnp.int32, (128, 2), 0) // IMG
    hc = lax.broadcasted_iota(jnp.int32, (128, 2), 1)
    h = (hr == hc).astype(jnp.float32)
    res = pl.pallas_call(
        _tc_body,
        grid=(NU // NB,),
        in_specs=[
            pl.BlockSpec((IMG, 128), lambda i: (0, 0)),
            pl.BlockSpec((128, 2), lambda i: (0, 0)),
            pl.BlockSpec((NB * IMG, SHEET), lambda i: (i, 0)),
            pl.BlockSpec((NB * IMG, SHEET), lambda i: (i, 0)),
        ],
        out_specs=pl.BlockSpec(memory_space=pltpu.SMEM),
        out_shape=jax.ShapeDtypeStruct((1, 1), jnp.float32),
        scratch_shapes=[pltpu.VMEM((NSL * NB, 128), jnp.float32)],
    )(x2, h, som, running_variance)
    return res[0, 0]
